# baseline (speedup 1.0000x reference)
import functools

import jax
import jax.numpy as jnp
from jax import lax
from jax.experimental import pallas as pl
from jax.experimental.pallas import tpu as pltpu
from jax.experimental.pallas import tpu_sc as plsc

N_TOK = 32768
D_EMB = 512
S_DIM = 256

NC = 2
NS = 16
NW = NC * NS
B_PER_W = N_TOK // NW
CH = 64
N_CHUNK = B_PER_W // CH

_sc_mesh = plsc.VectorSubcoreMesh(core_axis_name="c", subcore_axis_name="s")


@functools.partial(
    pl.kernel,
    out_type=jax.ShapeDtypeStruct((N_TOK, D_EMB), jnp.float32),
    mesh=_sc_mesh,
    scratch_types=[
        pltpu.VMEM((N_CHUNK, CH), jnp.int32),
        pltpu.VMEM((CH, D_EMB), jnp.float32),
        pltpu.VMEM((CH, D_EMB), jnp.float32),
        pltpu.SemaphoreType.DMA,
        pltpu.SemaphoreType.DMA,
        pltpu.SemaphoreType.DMA,
    ],
)
def _sc_gather(word_hbm, table_hbm, out_hbm, idx_v, rows_a, rows_b, isem, gsem, ssem):
    wid = lax.axis_index("s") * NC + lax.axis_index("c")
    base = wid * B_PER_W
    idx_loads = [
        pltpu.async_copy(word_hbm.at[pl.ds(base + c * CH, CH)], idx_v.at[c], isem)
        for c in range(N_CHUNK)
    ]
    bufs = (rows_a, rows_b)
    nb = len(bufs)
    gathers = [None] * N_CHUNK
    stores = [None] * N_CHUNK
    for c in range(min(nb, N_CHUNK)):
        idx_loads[c].wait()
        gathers[c] = pltpu.async_copy(
            table_hbm.at[idx_v.at[c]], bufs[c % nb], gsem
        )
    for c in range(N_CHUNK):
        buf = bufs[c % nb]
        gathers[c].wait()
        stores[c] = pltpu.async_copy(
            buf, out_hbm.at[pl.ds(base + c * CH, CH)], ssem
        )
        if c + nb < N_CHUNK:
            stores[c].wait()
            idx_loads[c + nb].wait()
            gathers[c + nb] = pltpu.async_copy(
                table_hbm.at[idx_v.at[c + nb]], buf, gsem
            )
    for c in range(max(0, N_CHUNK - nb), N_CHUNK):
        stores[c].wait()


BM = 4096
KS = D_EMB // 2


def _mlp_body(x1_ref, x2_ref, w1_ref, w2_ref, b_ref, o_ref):
    acc = jnp.dot(
        x1_ref[...].astype(jnp.bfloat16),
        w1_ref[...].astype(jnp.bfloat16),
        preferred_element_type=jnp.float32,
    )
    acc = acc + jnp.dot(
        x2_ref[...].astype(jnp.bfloat16),
        w2_ref[...].astype(jnp.bfloat16),
        preferred_element_type=jnp.float32,
    )
    o_ref[...] = jnp.tanh(acc + b_ref[...])


_tc_mlp = pl.pallas_call(
    _mlp_body,
    grid=(N_TOK // BM,),
    in_specs=[
        pl.BlockSpec((BM, KS), lambda i: (i, 0)),
        pl.BlockSpec((BM, KS), lambda i: (i, 1)),
        pl.BlockSpec((KS, S_DIM), lambda i: (0, 0)),
        pl.BlockSpec((KS, S_DIM), lambda i: (1, 0)),
        pl.BlockSpec((1, S_DIM), lambda i: (0, 0)),
    ],
    out_specs=pl.BlockSpec((BM, S_DIM), lambda i: (i, 0)),
    out_shape=jax.ShapeDtypeStruct((N_TOK, S_DIM), jnp.float32),
)


def kernel(word, word_embeddings, grammar_preterminates, W1, b1):
    del grammar_preterminates
    emb = _sc_gather(word.astype(jnp.int32), word_embeddings)
    return _tc_mlp(emb, emb, W1, W1, b1.reshape(1, S_DIM))
